# baseline (speedup 1.0000x reference)
import jax
import jax.numpy as jnp
from jax import lax
from jax.experimental import pallas as pl
from jax.experimental.pallas import tpu as pltpu

_TILE = 512
_K = 64


def _scorer_body(x_ref, w1_ref, b1_ref, w2_ref, out_ref):
    x = x_ref[...].astype(jnp.bfloat16)
    w1 = w1_ref[...].astype(jnp.bfloat16)
    h = lax.dot_general(
        x, w1, (((1,), (1,)), ((), ())),
        preferred_element_type=jnp.float32,
    )
    h = jnp.maximum(h + b1_ref[...], 0.0).astype(jnp.bfloat16)
    w2 = w2_ref[...].astype(jnp.bfloat16)
    s = lax.dot_general(
        h, w2, (((1,), (1,)), ((), ())),
        preferred_element_type=jnp.float32,
    )
    out_ref[0, 0, :] = s[:, 0]


def _topk_body(s_ref, am_ref, o_ref):
    s = s_ref[...]
    am = am_ref[...]
    neg = jnp.float32(-jnp.inf)
    s = jnp.where(am != 0, s, neg)
    iota = lax.broadcasted_iota(jnp.int32, s.shape, 1)

    def step(_, sel):
        unsel = sel == 0
        active = jnp.where(unsel, s, neg)
        mx = jnp.max(active, axis=1, keepdims=True)
        cand = (active == mx) & unsel
        first = jnp.min(
            jnp.where(cand, iota, s.shape[1]), axis=1, keepdims=True
        )
        return jnp.where(iota == first, 1, sel)

    sel = lax.fori_loop(0, _K, step, jnp.zeros(s.shape, jnp.int32))
    o_ref[...] = jnp.where(iota == 0, 1, sel)


def _scores(x, W1, b1, W2, interpret=False):
    n_rows, H = x.shape
    H2 = W1.shape[0]
    n_tiles = n_rows // _TILE
    out = pl.pallas_call(
        _scorer_body,
        grid=(n_tiles,),
        in_specs=[
            pl.BlockSpec((_TILE, H), lambda i: (i, 0)),
            pl.BlockSpec((H2, H), lambda i: (0, 0)),
            pl.BlockSpec((1, H2), lambda i: (0, 0)),
            pl.BlockSpec((128, H2), lambda i: (0, 0)),
        ],
        out_specs=pl.BlockSpec((1, 1, _TILE), lambda i: (i, 0, 0)),
        out_shape=jax.ShapeDtypeStruct((n_tiles, 1, _TILE), jnp.float32),
        interpret=interpret,
    )(x, W1, b1.reshape(1, H2), jnp.pad(W2, ((0, 127), (0, 0))))
    return out.reshape(n_rows)


def kernel(hidden_states, attention_mask, W1, b1, W2, b2):
    B, S = attention_mask.shape
    H = hidden_states.shape[-1]
    x = hidden_states.reshape(B * S, H)
    scores = _scores(x, W1, b1, W2).reshape(B, S)
    mask = pl.pallas_call(
        _topk_body,
        out_shape=jax.ShapeDtypeStruct((B, S), jnp.int32),
    )(scores, attention_mask)
    return mask
